# baseline (speedup 1.0000x reference)
import jax
import jax.numpy as jnp
from jax import lax
from jax.experimental import pallas as pl
from jax.experimental.pallas import tpu as pltpu
from jax.experimental.pallas import tpu_sc as plsc

B = 16384
N_ROWS = 1000000
D_MODEL = 64
PAIR = 2 * D_MODEL
STRIDE1 = 1000.0

NC = 2
NS = 16
L = 16
NW = NC * NS
B_PER_W = B // NW
CHUNK = 128
N_CHUNKS = B_PER_W // CHUNK
GROUPS = CHUNK // L
D_REGS = D_MODEL // L


def _sc_body(c0_hbm, c1_hbm, table_hbm, out_hbm,
             c0_v, c1_v, jq_v, bq_v, pairs_v, rows_v, sem):
    wid = lax.axis_index("s") * NC + lax.axis_index("c")
    base = wid * B_PER_W

    pltpu.sync_copy(c0_hbm.at[pl.ds(base, B_PER_W)], c0_v)
    pltpu.sync_copy(c1_hbm.at[pl.ds(base, B_PER_W)], c1_v)

    def make_extract(c):
        def extract(i, _):
            h = bq_v[pl.ds(c * CHUNK + i, L)][0]
            for k in range(D_REGS):
                rows_v[c * CHUNK + i, pl.ds(k * L, L)] = (
                    pairs_v[c % 2, i, pl.ds(h + k * L, L)]
                )
            return 0
        return extract

    desc = [None] * N_CHUNKS
    for c in range(N_CHUNKS):
        for g in range(GROUPS):
            off = c * CHUNK + g * L
            v0 = c0_v[pl.ds(off, L)]
            v1 = c1_v[pl.ds(off, L)]
            idx = (v0 + STRIDE1 * v1).astype(jnp.int32)
            jq_v[c, pl.ds(g * L, L)] = lax.shift_right_logical(idx, 1)
            bq_v[pl.ds(off, L)] = lax.bitwise_and(idx, 1) * D_MODEL
        desc[c] = pltpu.async_copy(
            table_hbm.at[jq_v.at[c]], pairs_v.at[c % 2], sem
        )
        if c >= 1:
            desc[c - 1].wait()
            lax.fori_loop(0, CHUNK, make_extract(c - 1), 0)
    desc[N_CHUNKS - 1].wait()
    lax.fori_loop(0, CHUNK, make_extract(N_CHUNKS - 1), 0)

    pltpu.sync_copy(rows_v, out_hbm.at[pl.ds(base, B_PER_W)])


@jax.jit
def kernel(x_entity0, embeddings):
    mesh = plsc.VectorSubcoreMesh(core_axis_name="c", subcore_axis_name="s")
    run = pl.kernel(
        _sc_body,
        out_type=jax.ShapeDtypeStruct((B, D_MODEL), jnp.float32),
        mesh=mesh,
        scratch_types=[
            pltpu.VMEM((B_PER_W,), jnp.float32),
            pltpu.VMEM((B_PER_W,), jnp.float32),
            pltpu.VMEM((N_CHUNKS, CHUNK), jnp.int32),
            pltpu.VMEM((B_PER_W + L,), jnp.int32),
            pltpu.VMEM((2, CHUNK, PAIR), jnp.float32),
            pltpu.VMEM((B_PER_W, D_MODEL), jnp.float32),
            pltpu.SemaphoreType.DMA,
        ],
    )
    zero = x_entity0[0, 0] * jnp.float32(0.0)
    table = embeddings.reshape(N_ROWS // 2, PAIR) + zero
    return run(x_entity0[:, 0], x_entity0[:, 1], table)


# speedup vs baseline: 2.1665x; 2.1665x over previous
import jax
import jax.numpy as jnp
from jax import lax
from jax.experimental import pallas as pl
from jax.experimental.pallas import tpu as pltpu
from jax.experimental.pallas import tpu_sc as plsc

B = 16384
D_MODEL = 64
STRIDE1 = 1000.0

NC = 2
NS = 16
L = 16
NW = NC * NS
B_PER_W = B // NW
GROUPS = B_PER_W // L
CHUNK = 64
N_CHUNKS = B_PER_W // CHUNK


def _sc_body(c0_hbm, c1_hbm, table_hbm, out_hbm,
             c0_v, c1_v, iq_v, rows_v, sem):
    wid = lax.axis_index("s") * NC + lax.axis_index("c")
    base = wid * B_PER_W

    pltpu.sync_copy(c0_hbm.at[pl.ds(base, B_PER_W)], c0_v)
    pltpu.sync_copy(c1_hbm.at[pl.ds(base, B_PER_W)], c1_v)

    for g in range(GROUPS):
        v0 = c0_v[pl.ds(g * L, L)]
        v1 = c1_v[pl.ds(g * L, L)]
        iq_v[pl.ds(g * L, L)] = (v0 + STRIDE1 * v1).astype(jnp.int32)

    def fire(g, _):
        vec = iq_v[pl.ds(g * L, L)]
        for j in range(L):
            pltpu.async_copy(table_hbm.at[vec[j]], rows_v.at[g * L + j], sem)
        return 0

    lax.fori_loop(0, GROUPS, fire, 0)
    pltpu.make_async_copy(
        out_hbm.at[pl.ds(base, B_PER_W)],
        rows_v,
        sem,
    ).wait()

    pltpu.sync_copy(rows_v, out_hbm.at[pl.ds(base, B_PER_W)])


@jax.jit
def kernel(x_entity0, embeddings):
    mesh = plsc.VectorSubcoreMesh(core_axis_name="c", subcore_axis_name="s")
    run = pl.kernel(
        _sc_body,
        out_type=jax.ShapeDtypeStruct((B, D_MODEL), jnp.float32),
        mesh=mesh,
        scratch_types=[
            pltpu.VMEM((B_PER_W,), jnp.float32),
            pltpu.VMEM((B_PER_W,), jnp.float32),
            pltpu.VMEM((B_PER_W + L,), jnp.int32),
            pltpu.VMEM((B_PER_W, D_MODEL), jnp.float32),
            pltpu.SemaphoreType.DMA,
        ],
    )
    return run(x_entity0[:, 0], x_entity0[:, 1], embeddings)
